# baseline (speedup 1.0000x reference)
import jax
import jax.numpy as jnp
from jax.experimental import pallas as pl
from jax.experimental.pallas import tpu as pltpu

NHEADS = 12
NPTS = 8
HDIM = 64
CHUNK = 256


def _fused_kernel(q_ref, rp_ref, wcat_ref, bcat_ref, wout_ref, bout_ref, out_ref):
    h = pl.program_id(1)
    L, E = q_ref.shape[1], q_ref.shape[2]
    q = q_ref[0]
    r = jnp.dot(q, wcat_ref[0], preferred_element_type=jnp.float32) + bcat_ref[0]
    val = r[:, 0:HDIM]
    xf = r[:, HDIM:HDIM + NPTS] + rp_ref[0, :, 0:1]
    yf = r[:, HDIM + NPTS:HDIM + 2 * NPTS] + rp_ref[0, :, 1:2]
    logits = r[:, HDIM + 2 * NPTS:HDIM + 3 * NPTS]
    m = jnp.max(logits, axis=1, keepdims=True)
    e = jnp.exp(logits - m)
    attn = e / jnp.sum(e, axis=1, keepdims=True)

    gw = 32
    i = jax.lax.broadcasted_iota(jnp.int32, (1, CHUNK), 1)
    xg = (i % gw).astype(jnp.float32)
    txa_list = []
    for p in range(NPTS):
        ap = attn[:, p:p + 1]
        dx = jnp.abs(xg - xf[:, p:p + 1])
        txa_list.append(jnp.maximum(ap - ap * dx, 0.0))
    sampled = jnp.zeros((L, HDIM), jnp.float32)
    for c in range(L // CHUNK):
        yg = (i // gw + c * (CHUNK // gw)).astype(jnp.float32)
        acc = None
        for p in range(NPTS):
            dy = jnp.abs(yg - yf[:, p:p + 1])
            ty = jnp.maximum(1.0 - dy, 0.0)
            term = txa_list[p] * ty
            acc = term if acc is None else acc + term
        sampled = sampled + jnp.dot(acc, val[c * CHUNK:(c + 1) * CHUNK, :],
                                    preferred_element_type=jnp.float32)

    contrib = jnp.dot(sampled, wout_ref[0], preferred_element_type=jnp.float32)

    @pl.when(h == 0)
    def _():
        out_ref[0] = contrib + bout_ref[...]

    @pl.when(h != 0)
    def _():
        out_ref[0] = out_ref[0] + contrib


def kernel(query, reference_points, W_off, b_off, W_attn, b_attn, W_val, b_val, W_out, b_out):
    N, H, W, E = query.shape
    L = H * W
    qf = query.reshape(N, L, E)
    Wv = W_val.reshape(E, NHEADS, HDIM).transpose(1, 0, 2)
    Wo2 = W_off.reshape(E, NHEADS, NPTS, 2)
    Wox = float(W) * Wo2[..., 0].transpose(1, 0, 2)
    Woy = float(H) * Wo2[..., 1].transpose(1, 0, 2)
    Wa = W_attn.reshape(E, NHEADS, NPTS).transpose(1, 0, 2)
    Wcat = jnp.concatenate([Wv, Wox, Woy, Wa], axis=2)
    bo2 = b_off.reshape(NHEADS, NPTS, 2)
    bcat = jnp.concatenate([b_val.reshape(NHEADS, HDIM),
                            float(W) * bo2[..., 0], float(H) * bo2[..., 1],
                            b_attn.reshape(NHEADS, NPTS)], axis=1)[:, None, :]
    rp = reference_points.reshape(N, L, 2) * jnp.array([W, H], jnp.float32) - 0.5
    Wob = W_out.reshape(NHEADS, HDIM, E)
    bob = b_out.reshape(1, E)

    out = pl.pallas_call(
        _fused_kernel,
        grid=(N, NHEADS),
        in_specs=[
            pl.BlockSpec((1, L, E), lambda n, h: (n, 0, 0)),
            pl.BlockSpec((1, L, 2), lambda n, h: (n, 0, 0)),
            pl.BlockSpec((1, E, HDIM + 3 * NPTS), lambda n, h: (h, 0, 0)),
            pl.BlockSpec((1, 1, HDIM + 3 * NPTS), lambda n, h: (h, 0, 0)),
            pl.BlockSpec((1, HDIM, E), lambda n, h: (h, 0, 0)),
            pl.BlockSpec((1, E), lambda n, h: (0, 0)),
        ],
        out_specs=pl.BlockSpec((1, L, E), lambda n, h: (n, 0, 0)),
        out_shape=jax.ShapeDtypeStruct((N, L, E), jnp.float32),
        compiler_params=pltpu.CompilerParams(
            dimension_semantics=("parallel", "arbitrary")),
    )(qf, rp, Wcat, bcat, Wob, bob)
    return out.reshape(N, H, W, E)


# speedup vs baseline: 1.3786x; 1.3786x over previous
import jax
import jax.numpy as jnp
from jax.experimental import pallas as pl
from jax.experimental.pallas import tpu as pltpu

NHEADS = 12
NPTS = 8
HDIM = 64
CHUNK = 256


def _fused_kernel(qt_ref, rpt_ref, wcat_ref, bcat_ref, wout_ref, bout_ref, out_ref):
    h = pl.program_id(1)
    L = qt_ref.shape[2]
    qt = qt_ref[0]
    rt = jnp.dot(wcat_ref[0], qt, preferred_element_type=jnp.float32) + bcat_ref[0]
    valt = rt[0:HDIM, :]
    xft = rt[HDIM:HDIM + NPTS, :] + rpt_ref[0, 0:1, :]
    yft = rt[HDIM + NPTS:HDIM + 2 * NPTS, :] + rpt_ref[0, 1:2, :]
    logits = rt[HDIM + 2 * NPTS:HDIM + 3 * NPTS, :]
    m = jnp.max(logits, axis=0, keepdims=True)
    e = jnp.exp(logits - m)
    attnt = e / jnp.sum(e, axis=0, keepdims=True)

    gw = 32
    rr = jax.lax.broadcasted_iota(jnp.int32, (CHUNK, 1), 0)
    xg = (rr % gw).astype(jnp.float32)
    sampledt = jnp.zeros((HDIM, L), jnp.float32)
    for c in range(L // CHUNK):
        yg = (rr // gw + c * (CHUNK // gw)).astype(jnp.float32)
        acc = None
        for p in range(NPTS):
            ap = attnt[p:p + 1, :]
            dx = jnp.abs(xg - xft[p:p + 1, :])
            txa = jnp.maximum(ap - ap * dx, 0.0)
            dy = jnp.abs(yg - yft[p:p + 1, :])
            ty = jnp.maximum(1.0 - dy, 0.0)
            term = txa * ty
            acc = term if acc is None else acc + term
        sampledt = sampledt + jnp.dot(valt[:, c * CHUNK:(c + 1) * CHUNK], acc,
                                      preferred_element_type=jnp.float32)

    contrib = jnp.dot(wout_ref[0], sampledt, preferred_element_type=jnp.float32)

    @pl.when(h == 0)
    def _():
        out_ref[0] = contrib + bout_ref[...]

    @pl.when(h != 0)
    def _():
        out_ref[0] = out_ref[0] + contrib


def kernel(query, reference_points, W_off, b_off, W_attn, b_attn, W_val, b_val, W_out, b_out):
    N, H, W, E = query.shape
    L = H * W
    qt = query.reshape(N, L, E).transpose(0, 2, 1)
    Wv = W_val.reshape(E, NHEADS, HDIM).transpose(1, 0, 2)
    Wo2 = W_off.reshape(E, NHEADS, NPTS, 2)
    Wox = float(W) * Wo2[..., 0].transpose(1, 0, 2)
    Woy = float(H) * Wo2[..., 1].transpose(1, 0, 2)
    Wa = W_attn.reshape(E, NHEADS, NPTS).transpose(1, 0, 2)
    Wcat = jnp.concatenate([Wv, Wox, Woy, Wa], axis=2)
    WcatT = Wcat.transpose(0, 2, 1)
    bo2 = b_off.reshape(NHEADS, NPTS, 2)
    bcat = jnp.concatenate([b_val.reshape(NHEADS, HDIM),
                            float(W) * bo2[..., 0], float(H) * bo2[..., 1],
                            b_attn.reshape(NHEADS, NPTS)], axis=1)[:, :, None]
    rpt = (reference_points.reshape(N, L, 2) * jnp.array([W, H], jnp.float32)
           - 0.5).transpose(0, 2, 1)
    WoutT = W_out.reshape(NHEADS, HDIM, E).transpose(0, 2, 1)
    boutT = b_out.reshape(E, 1)

    outT = pl.pallas_call(
        _fused_kernel,
        grid=(N, NHEADS),
        in_specs=[
            pl.BlockSpec((1, E, L), lambda n, h: (n, 0, 0)),
            pl.BlockSpec((1, 2, L), lambda n, h: (n, 0, 0)),
            pl.BlockSpec((1, HDIM + 3 * NPTS, E), lambda n, h: (h, 0, 0)),
            pl.BlockSpec((1, HDIM + 3 * NPTS, 1), lambda n, h: (h, 0, 0)),
            pl.BlockSpec((1, E, HDIM), lambda n, h: (h, 0, 0)),
            pl.BlockSpec((E, 1), lambda n, h: (0, 0)),
        ],
        out_specs=pl.BlockSpec((1, E, L), lambda n, h: (n, 0, 0)),
        out_shape=jax.ShapeDtypeStruct((N, E, L), jnp.float32),
        compiler_params=pltpu.CompilerParams(
            dimension_semantics=("parallel", "arbitrary")),
    )(qt, rpt, WcatT, bcat, WoutT, boutT)
    return outT.transpose(0, 2, 1).reshape(N, H, W, E)


# speedup vs baseline: 2.6365x; 1.9124x over previous
import jax
import jax.numpy as jnp
from jax.experimental import pallas as pl
from jax.experimental.pallas import tpu as pltpu

NHEADS = 12
NPTS = 8
HDIM = 64
CHUNK = 256


def _fused_kernel(qt_ref, rpt_ref, wcat_ref, bcat_ref, wout_ref, bout_ref, out_ref):
    h = pl.program_id(1)
    L = qt_ref.shape[2]
    qt = qt_ref[0]
    rt = jnp.dot(wcat_ref[0], qt, preferred_element_type=jnp.float32) + bcat_ref[0]
    valt = rt[0:HDIM, :]
    xft = rt[HDIM:HDIM + NPTS, :] + rpt_ref[0, 0:1, :]
    yft = rt[HDIM + NPTS:HDIM + 2 * NPTS, :] + rpt_ref[0, 1:2, :]
    logits = rt[HDIM + 2 * NPTS:HDIM + 3 * NPTS, :]
    m = jnp.max(logits, axis=0, keepdims=True)
    e = jnp.exp(logits - m)
    attnt = e / jnp.sum(e, axis=0, keepdims=True)

    gw = 32
    g = jax.lax.broadcasted_iota(jnp.int32, (gw, 1), 0).astype(jnp.float32)
    txa_list = []
    ty_list = []
    for p in range(NPTS):
        ap = attnt[p:p + 1, :]
        txa_list.append(jnp.maximum(ap - ap * jnp.abs(g - xft[p:p + 1, :]), 0.0))
        ty_list.append(jnp.maximum(1.0 - jnp.abs(g - yft[p:p + 1, :]), 0.0))
    sampledt = jnp.zeros((HDIM, L), jnp.float32)
    for c in range(L // CHUNK):
        slabs = []
        for j in range(CHUNK // gw):
            y = c * (CHUNK // gw) + j
            s = None
            for p in range(NPTS):
                t = txa_list[p] * ty_list[p][y:y + 1, :]
                s = t if s is None else s + t
            slabs.append(s)
        acc = jnp.concatenate(slabs, axis=0)
        sampledt = sampledt + jnp.dot(valt[:, c * CHUNK:(c + 1) * CHUNK], acc,
                                      preferred_element_type=jnp.float32)

    contrib = jnp.dot(wout_ref[0], sampledt, preferred_element_type=jnp.float32)

    @pl.when(h == 0)
    def _():
        out_ref[0] = contrib + bout_ref[...]

    @pl.when(h != 0)
    def _():
        out_ref[0] = out_ref[0] + contrib


def kernel(query, reference_points, W_off, b_off, W_attn, b_attn, W_val, b_val, W_out, b_out):
    N, H, W, E = query.shape
    L = H * W
    qt = query.reshape(N, L, E).transpose(0, 2, 1)
    Wv = W_val.reshape(E, NHEADS, HDIM).transpose(1, 0, 2)
    Wo2 = W_off.reshape(E, NHEADS, NPTS, 2)
    Wox = float(W) * Wo2[..., 0].transpose(1, 0, 2)
    Woy = float(H) * Wo2[..., 1].transpose(1, 0, 2)
    Wa = W_attn.reshape(E, NHEADS, NPTS).transpose(1, 0, 2)
    Wcat = jnp.concatenate([Wv, Wox, Woy, Wa], axis=2)
    WcatT = Wcat.transpose(0, 2, 1)
    bo2 = b_off.reshape(NHEADS, NPTS, 2)
    bcat = jnp.concatenate([b_val.reshape(NHEADS, HDIM),
                            float(W) * bo2[..., 0], float(H) * bo2[..., 1],
                            b_attn.reshape(NHEADS, NPTS)], axis=1)[:, :, None]
    rpt = (reference_points.reshape(N, L, 2) * jnp.array([W, H], jnp.float32)
           - 0.5).transpose(0, 2, 1)
    WoutT = W_out.reshape(NHEADS, HDIM, E).transpose(0, 2, 1)
    boutT = b_out.reshape(E, 1)

    outT = pl.pallas_call(
        _fused_kernel,
        grid=(N, NHEADS),
        in_specs=[
            pl.BlockSpec((1, E, L), lambda n, h: (n, 0, 0)),
            pl.BlockSpec((1, 2, L), lambda n, h: (n, 0, 0)),
            pl.BlockSpec((1, HDIM + 3 * NPTS, E), lambda n, h: (h, 0, 0)),
            pl.BlockSpec((1, HDIM + 3 * NPTS, 1), lambda n, h: (h, 0, 0)),
            pl.BlockSpec((1, E, HDIM), lambda n, h: (h, 0, 0)),
            pl.BlockSpec((E, 1), lambda n, h: (0, 0)),
        ],
        out_specs=pl.BlockSpec((1, E, L), lambda n, h: (n, 0, 0)),
        out_shape=jax.ShapeDtypeStruct((N, E, L), jnp.float32),
        compiler_params=pltpu.CompilerParams(
            dimension_semantics=("parallel", "arbitrary")),
    )(qt, rpt, WcatT, bcat, WoutT, boutT)
    return outT.transpose(0, 2, 1).reshape(N, H, W, E)
